# baseline (speedup 1.0000x reference)
import functools

import jax
import jax.numpy as jnp
import numpy as np
from jax import lax
from jax.experimental import pallas as pl
from jax.experimental.pallas import tpu as pltpu
from jax.experimental.pallas import tpu_sc as plsc

_CUTOFF = 5.0
_NRBF = 16
_INV_W = np.float32(1.0) / np.float32(_CUTOFF / (_NRBF - 1))


_L = 64000


def _rbf_body(d3_ref, d128_ref, fij_ref, fcut_ref):
    d = d3_ref[...].reshape(1, _L)
    db = jnp.broadcast_to(d * _INV_W, (_NRBF, _L))
    k = lax.broadcasted_iota(jnp.int32, (_NRBF, _L), 0).astype(jnp.float32)
    t = db - k
    fij_ref[...] = jnp.exp(-0.5 * t * t)
    x = d128_ref[...]
    fc = 0.5 * (jnp.cos(x * np.float32(np.pi / _CUTOFF)) + 1.0)
    fcut_ref[...] = jnp.where(x < _CUTOFF, fc, 0.0)


def _rbf_tc(d_flat):
    p = d_flat.shape[0]
    grid = p // _L
    r2 = _L // 128

    d3 = d_flat.reshape(grid, 1, _L)
    d128 = d_flat.reshape(grid, r2, 128)

    fij_t, fcut = pl.pallas_call(
        _rbf_body,
        grid=(grid,),
        in_specs=[
            pl.BlockSpec((1, 1, _L), lambda i: (i, 0, 0)),
            pl.BlockSpec((1, r2, 128), lambda i: (i, 0, 0)),
        ],
        out_specs=[
            pl.BlockSpec((_NRBF, _L), lambda i: (0, i)),
            pl.BlockSpec((1, r2, 128), lambda i: (i, 0, 0)),
        ],
        out_shape=[
            jax.ShapeDtypeStruct((_NRBF, p), jnp.float32),
            jax.ShapeDtypeStruct((grid, r2, 128), jnp.float32),
        ],
        compiler_params=pltpu.CompilerParams(
            dimension_semantics=("arbitrary",)),
    )(d3, d128)
    return fij_t.T, fcut.reshape(p, 1)


_SC_NW = 32
_SC_BPW = 3200
_SC_GRP = _SC_BPW // 16
_N_ATOMS = 100000
_EMB_D = 32
_MAX_Z = 101
_SC_NPAD = _SC_NW * _SC_BPW


@functools.cache
def _build_emb_sc():
    @functools.partial(
        pl.kernel,
        out_type=jax.ShapeDtypeStruct((_EMB_D, _SC_NPAD), jnp.float32),
        mesh=plsc.VectorSubcoreMesh(core_axis_name="c", subcore_axis_name="s"),
        compiler_params=pltpu.CompilerParams(needs_layout_passes=False),
        scratch_types=[
            pltpu.VMEM((_SC_BPW,), jnp.int32),
            pltpu.VMEM((_MAX_Z * _EMB_D,), jnp.float32),
            pltpu.VMEM((_EMB_D, _SC_BPW), jnp.float32),
        ],
    )
    def _emb_sc(idx_hbm, table_hbm, out_hbm, idx_v, table_v, rows_v):
        wid = lax.axis_index("s") * 2 + lax.axis_index("c")
        base = wid * _SC_BPW
        pltpu.sync_copy(idx_hbm.at[pl.ds(base, _SC_BPW)], idx_v)
        pltpu.sync_copy(table_hbm, table_v)

        @plsc.parallel_loop(0, _SC_GRP, step=1, unroll=4)
        def _gather_loop(g):
            src = idx_v[pl.ds(g * 16, 16)] * _EMB_D
            for j in range(_EMB_D):
                vals = plsc.load_gather(table_v, [src + j])
                rows_v[j, pl.ds(g * 16, 16)] = vals
        pltpu.sync_copy(rows_v, out_hbm.at[:, pl.ds(base, _SC_BPW)])

    return _emb_sc


def _emb_gather(atomic_numbers, embedding_table):
    pad = _SC_NPAD - atomic_numbers.shape[0]
    idx = jnp.concatenate([atomic_numbers, jnp.zeros((pad,), jnp.int32)])
    emb_t = _build_emb_sc()(idx, embedding_table.reshape(-1))
    return emb_t[:, :_N_ATOMS].T


def kernel(d_ij, atomic_numbers, embedding_table):
    f_ij, f_cutoff = _rbf_tc(d_ij.reshape(-1))
    atomic_embedding = _emb_gather(atomic_numbers, embedding_table)
    return (f_ij, f_cutoff, atomic_embedding)


# speedup vs baseline: 2.1533x; 2.1533x over previous
import functools

import jax
import jax.numpy as jnp
import numpy as np
from jax import lax
from jax.experimental import pallas as pl
from jax.experimental.pallas import tpu as pltpu
from jax.experimental.pallas import tpu_sc as plsc

_CUTOFF = 5.0
_NRBF = 16
_INV_W = np.float32(1.0) / np.float32(_CUTOFF / (_NRBF - 1))


_L = 128000


def _rbf_body(d3_ref, d128_ref, fij_ref, fcut_ref):
    d = d3_ref[...].reshape(1, _L)
    db = jnp.broadcast_to(d * _INV_W, (_NRBF, _L))
    k = lax.broadcasted_iota(jnp.int32, (_NRBF, _L), 0).astype(jnp.float32)
    t = db - k
    fij_ref[...] = jnp.exp(-0.5 * t * t)
    x = d128_ref[...]
    fc = 0.5 * (jnp.cos(x * np.float32(np.pi / _CUTOFF)) + 1.0)
    fcut_ref[...] = jnp.where(x < _CUTOFF, fc, 0.0)


def _rbf_tc(d_flat):
    p = d_flat.shape[0]
    grid = p // _L
    r2 = _L // 128

    d3 = d_flat.reshape(grid, 1, _L)
    d128 = d_flat.reshape(grid, r2, 128)

    fij_t, fcut = pl.pallas_call(
        _rbf_body,
        grid=(grid,),
        in_specs=[
            pl.BlockSpec((1, 1, _L), lambda i: (i, 0, 0)),
            pl.BlockSpec((1, r2, 128), lambda i: (i, 0, 0)),
        ],
        out_specs=[
            pl.BlockSpec((_NRBF, _L), lambda i: (0, i)),
            pl.BlockSpec((1, r2, 128), lambda i: (i, 0, 0)),
        ],
        out_shape=[
            jax.ShapeDtypeStruct((_NRBF, p), jnp.float32),
            jax.ShapeDtypeStruct((grid, r2, 128), jnp.float32),
        ],
        compiler_params=pltpu.CompilerParams(
            dimension_semantics=("arbitrary",)),
    )(d3, d128)
    return fij_t.T, fcut.reshape(p, 1)


_SC_NW = 32
_SC_BPW = 3200
_SC_GRP = _SC_BPW // 16
_N_ATOMS = 100000
_EMB_D = 32
_MAX_Z = 101
_SC_NPAD = _SC_NW * _SC_BPW


@functools.cache
def _build_emb_sc():
    @functools.partial(
        pl.kernel,
        out_type=jax.ShapeDtypeStruct((_EMB_D, _SC_NPAD), jnp.float32),
        mesh=plsc.VectorSubcoreMesh(core_axis_name="c", subcore_axis_name="s"),
        compiler_params=pltpu.CompilerParams(needs_layout_passes=False),
        scratch_types=[
            pltpu.VMEM((_SC_BPW,), jnp.int32),
            pltpu.VMEM((_MAX_Z * _EMB_D,), jnp.float32),
            pltpu.VMEM((_EMB_D, _SC_BPW), jnp.float32),
        ],
    )
    def _emb_sc(idx_hbm, table_hbm, out_hbm, idx_v, table_v, rows_v):
        wid = lax.axis_index("s") * 2 + lax.axis_index("c")
        base = wid * _SC_BPW
        pltpu.sync_copy(idx_hbm.at[pl.ds(base, _SC_BPW)], idx_v)
        pltpu.sync_copy(table_hbm, table_v)

        @plsc.parallel_loop(0, _SC_GRP, step=1, unroll=4)
        def _gather_loop(g):
            src = idx_v[pl.ds(g * 16, 16)] * _EMB_D
            for j in range(_EMB_D):
                vals = plsc.load_gather(table_v, [src + j])
                rows_v[j, pl.ds(g * 16, 16)] = vals
        pltpu.sync_copy(rows_v, out_hbm.at[:, pl.ds(base, _SC_BPW)])

    return _emb_sc


def _emb_gather(atomic_numbers, embedding_table):
    pad = _SC_NPAD - atomic_numbers.shape[0]
    idx = jnp.concatenate([atomic_numbers, jnp.zeros((pad,), jnp.int32)])
    emb_t = _build_emb_sc()(idx, embedding_table.reshape(-1))
    return emb_t[:, :_N_ATOMS].T


def kernel(d_ij, atomic_numbers, embedding_table):
    f_ij, f_cutoff = _rbf_tc(d_ij.reshape(-1))
    atomic_embedding = _emb_gather(atomic_numbers, embedding_table)
    return (f_ij, f_cutoff, atomic_embedding)


# speedup vs baseline: 2.4038x; 1.1164x over previous
import functools

import jax
import jax.numpy as jnp
import numpy as np
from jax import lax
from jax.experimental import pallas as pl
from jax.experimental.pallas import tpu as pltpu
from jax.experimental.pallas import tpu_sc as plsc

_CUTOFF = 5.0
_NRBF = 16
_INV_W = np.float32(1.0) / np.float32(_CUTOFF / (_NRBF - 1))


_L = 128000

_Y_SCALE = np.float32(np.pi / (2.0 * _CUTOFF))
_ygrid = np.linspace(0.0, np.pi / 2, 4001)
_ugrid = _ygrid**2
_A = np.stack([_ugrid**i for i in range(6)], axis=1)
_CC = np.linalg.lstsq(_A, np.cos(_ygrid) ** 2, rcond=None)[0].astype(np.float32)


def _rbf_body(d3_ref, fij_ref, fcut_ref):
    d = d3_ref[...].reshape(1, _L)
    db = pltpu.repeat(d * _INV_W, _NRBF, 0)
    k = lax.broadcasted_iota(jnp.int32, (_NRBF, _L), 0).astype(jnp.float32)
    t = db - k
    fij_ref[...] = jnp.exp2(np.float32(-0.5 * np.log2(np.e)) * t * t)
    y = d * _Y_SCALE
    u = y * y
    fc = jnp.full_like(u, _CC[5])
    for c in _CC[4::-1]:
        fc = fc * u + c
    fcut_ref[...] = jnp.where(d < _CUTOFF, fc, 0.0).reshape(1, 1, _L)


def _rbf_tc(d_flat):
    p = d_flat.shape[0]
    grid = p // _L

    d3 = d_flat.reshape(grid, 1, _L)

    fij_t, fcut = pl.pallas_call(
        _rbf_body,
        grid=(grid,),
        in_specs=[
            pl.BlockSpec((1, 1, _L), lambda i: (i, 0, 0)),
        ],
        out_specs=[
            pl.BlockSpec((_NRBF, _L), lambda i: (0, i)),
            pl.BlockSpec((1, 1, _L), lambda i: (i, 0, 0)),
        ],
        out_shape=[
            jax.ShapeDtypeStruct((_NRBF, p), jnp.float32),
            jax.ShapeDtypeStruct((grid, 1, _L), jnp.float32),
        ],
        compiler_params=pltpu.CompilerParams(
            dimension_semantics=("arbitrary",)),
    )(d3)
    return fij_t.T, fcut.reshape(p, 1)


_SC_NW = 32
_SC_BPW = 3200
_SC_GRP = _SC_BPW // 16
_N_ATOMS = 100000
_EMB_D = 32
_MAX_Z = 101
_SC_NPAD = _SC_NW * _SC_BPW


@functools.cache
def _build_emb_sc():
    @functools.partial(
        pl.kernel,
        out_type=jax.ShapeDtypeStruct((_EMB_D, _SC_NPAD), jnp.float32),
        mesh=plsc.VectorSubcoreMesh(core_axis_name="c", subcore_axis_name="s"),
        compiler_params=pltpu.CompilerParams(needs_layout_passes=False),
        scratch_types=[
            pltpu.VMEM((_SC_BPW,), jnp.int32),
            pltpu.VMEM((_MAX_Z * _EMB_D,), jnp.float32),
            pltpu.VMEM((_EMB_D, _SC_BPW), jnp.float32),
        ],
    )
    def _emb_sc(idx_hbm, table_hbm, out_hbm, idx_v, table_v, rows_v):
        wid = lax.axis_index("s") * 2 + lax.axis_index("c")
        base = wid * _SC_BPW
        pltpu.sync_copy(idx_hbm.at[pl.ds(base, _SC_BPW)], idx_v)
        pltpu.sync_copy(table_hbm, table_v)

        @plsc.parallel_loop(0, _SC_GRP, step=1, unroll=4)
        def _gather_loop(g):
            src = idx_v[pl.ds(g * 16, 16)] * _EMB_D
            for j in range(_EMB_D):
                vals = plsc.load_gather(table_v, [src + j])
                rows_v[j, pl.ds(g * 16, 16)] = vals
        pltpu.sync_copy(rows_v, out_hbm.at[:, pl.ds(base, _SC_BPW)])

    return _emb_sc


def _emb_gather(atomic_numbers, embedding_table):
    pad = _SC_NPAD - atomic_numbers.shape[0]
    idx = jnp.concatenate([atomic_numbers, jnp.zeros((pad,), jnp.int32)])
    emb_t = _build_emb_sc()(idx, embedding_table.reshape(-1))
    return emb_t[:, :_N_ATOMS].T


def kernel(d_ij, atomic_numbers, embedding_table):
    f_ij, f_cutoff = _rbf_tc(d_ij.reshape(-1))
    atomic_embedding = _emb_gather(atomic_numbers, embedding_table)
    return (f_ij, f_cutoff, atomic_embedding)


# speedup vs baseline: 2.4187x; 1.0062x over previous
import functools

import jax
import jax.numpy as jnp
import numpy as np
from jax import lax
from jax.experimental import pallas as pl
from jax.experimental.pallas import tpu as pltpu
from jax.experimental.pallas import tpu_sc as plsc

_CUTOFF = 5.0
_NRBF = 16
_INV_W = np.float32(1.0) / np.float32(_CUTOFF / (_NRBF - 1))


_L = 160000

_Y_SCALE = np.float32(np.pi / (2.0 * _CUTOFF))
_ygrid = np.linspace(0.0, np.pi / 2, 4001)
_ugrid = _ygrid**2
_A = np.stack([_ugrid**i for i in range(6)], axis=1)
_CC = np.linalg.lstsq(_A, np.cos(_ygrid) ** 2, rcond=None)[0].astype(np.float32)


def _rbf_body(d3_ref, fij_ref, fcut_ref):
    d = d3_ref[...].reshape(1, _L)
    db = pltpu.repeat(d * _INV_W, _NRBF, 0)
    k = lax.broadcasted_iota(jnp.int32, (_NRBF, _L), 0).astype(jnp.float32)
    t = db - k
    fij_ref[...] = jnp.exp2(np.float32(-0.5 * np.log2(np.e)) * t * t)
    y = d * _Y_SCALE
    u = y * y
    fc = jnp.full_like(u, _CC[5])
    for c in _CC[4::-1]:
        fc = fc * u + c
    fcut_ref[...] = jnp.where(d < _CUTOFF, fc, 0.0).reshape(1, 1, _L)


def _rbf_tc(d_flat):
    p = d_flat.shape[0]
    grid = p // _L

    d3 = d_flat.reshape(grid, 1, _L)

    fij_t, fcut = pl.pallas_call(
        _rbf_body,
        grid=(grid,),
        in_specs=[
            pl.BlockSpec((1, 1, _L), lambda i: (i, 0, 0)),
        ],
        out_specs=[
            pl.BlockSpec((_NRBF, _L), lambda i: (0, i)),
            pl.BlockSpec((1, 1, _L), lambda i: (i, 0, 0)),
        ],
        out_shape=[
            jax.ShapeDtypeStruct((_NRBF, p), jnp.float32),
            jax.ShapeDtypeStruct((grid, 1, _L), jnp.float32),
        ],
        compiler_params=pltpu.CompilerParams(
            dimension_semantics=("arbitrary",)),
    )(d3)
    return fij_t.T, fcut.reshape(p, 1)


_SC_NW = 32
_SC_BPW = 3200
_SC_GRP = _SC_BPW // 16
_N_ATOMS = 100000
_EMB_D = 32
_MAX_Z = 101
_SC_NPAD = _SC_NW * _SC_BPW


@functools.cache
def _build_emb_sc():
    @functools.partial(
        pl.kernel,
        out_type=jax.ShapeDtypeStruct((_EMB_D, _SC_NPAD), jnp.float32),
        mesh=plsc.VectorSubcoreMesh(core_axis_name="c", subcore_axis_name="s"),
        compiler_params=pltpu.CompilerParams(needs_layout_passes=False),
        scratch_types=[
            pltpu.VMEM((_SC_BPW,), jnp.int32),
            pltpu.VMEM((_MAX_Z * _EMB_D,), jnp.float32),
            pltpu.VMEM((_EMB_D, _SC_BPW), jnp.float32),
        ],
    )
    def _emb_sc(idx_hbm, table_hbm, out_hbm, idx_v, table_v, rows_v):
        wid = lax.axis_index("s") * 2 + lax.axis_index("c")
        base = wid * _SC_BPW
        pltpu.sync_copy(idx_hbm.at[pl.ds(base, _SC_BPW)], idx_v)
        pltpu.sync_copy(table_hbm, table_v)

        @plsc.parallel_loop(0, _SC_GRP, step=1, unroll=4)
        def _gather_loop(g):
            src = idx_v[pl.ds(g * 16, 16)] * _EMB_D
            for j in range(_EMB_D):
                vals = plsc.load_gather(table_v, [src + j])
                rows_v[j, pl.ds(g * 16, 16)] = vals
        pltpu.sync_copy(rows_v, out_hbm.at[:, pl.ds(base, _SC_BPW)])

    return _emb_sc


def _emb_gather(atomic_numbers, embedding_table):
    pad = _SC_NPAD - atomic_numbers.shape[0]
    idx = jnp.concatenate([atomic_numbers, jnp.zeros((pad,), jnp.int32)])
    emb_t = _build_emb_sc()(idx, embedding_table.reshape(-1))
    return emb_t[:, :_N_ATOMS].T


def kernel(d_ij, atomic_numbers, embedding_table):
    f_ij, f_cutoff = _rbf_tc(d_ij.reshape(-1))
    atomic_embedding = _emb_gather(atomic_numbers, embedding_table)
    return (f_ij, f_cutoff, atomic_embedding)


# speedup vs baseline: 2.4238x; 1.0021x over previous
import functools

import jax
import jax.numpy as jnp
import numpy as np
from jax import lax
from jax.experimental import pallas as pl
from jax.experimental.pallas import tpu as pltpu
from jax.experimental.pallas import tpu_sc as plsc

_CUTOFF = 5.0
_NRBF = 16
_INV_W = np.float32(1.0) / np.float32(_CUTOFF / (_NRBF - 1))


_L = 160000

_Y_SCALE = np.float32(np.pi / (2.0 * _CUTOFF))
_ygrid = np.linspace(0.0, np.pi / 2, 4001)
_ugrid = _ygrid**2
_A = np.stack([_ugrid**i for i in range(6)], axis=1)
_CC = np.linalg.lstsq(_A, np.cos(_ygrid) ** 2, rcond=None)[0].astype(np.float32)


def _rbf_body(d3_ref, fij_ref, fcut_ref):
    d = d3_ref[...].reshape(1, _L)
    db = pltpu.repeat(d * _INV_W, _NRBF, 0)
    k = lax.broadcasted_iota(jnp.int32, (_NRBF, _L), 0).astype(jnp.float32)
    t = db - k
    fij_ref[...] = jnp.exp2(np.float32(-0.5 * np.log2(np.e)) * t * t)
    y = d * _Y_SCALE
    u = y * y
    fc = jnp.full_like(u, _CC[5])
    for c in _CC[4::-1]:
        fc = fc * u + c
    fcut_ref[...] = jnp.where(d < _CUTOFF, fc, 0.0).reshape(1, 1, _L)


def _rbf_tc(d_flat):
    p = d_flat.shape[0]
    grid = p // _L

    d3 = d_flat.reshape(grid, 1, _L)

    fij_t, fcut = pl.pallas_call(
        _rbf_body,
        grid=(grid,),
        in_specs=[
            pl.BlockSpec((1, 1, _L), lambda i: (i, 0, 0)),
        ],
        out_specs=[
            pl.BlockSpec((_NRBF, _L), lambda i: (0, i)),
            pl.BlockSpec((1, 1, _L), lambda i: (i, 0, 0)),
        ],
        out_shape=[
            jax.ShapeDtypeStruct((_NRBF, p), jnp.float32),
            jax.ShapeDtypeStruct((grid, 1, _L), jnp.float32),
        ],
        compiler_params=pltpu.CompilerParams(
            dimension_semantics=("arbitrary",)),
    )(d3)
    return fij_t.T, fcut.reshape(p, 1)


_SC_NW = 32
_SC_BPW = 3200
_SC_GRP = _SC_BPW // 16
_N_ATOMS = 100000
_EMB_D = 32
_MAX_Z = 101
_SC_NPAD = 100096
_SC_LAST = _SC_NPAD - (_SC_NW - 1) * _SC_BPW


@functools.cache
def _build_emb_sc():
    @functools.partial(
        pl.kernel,
        out_type=jax.ShapeDtypeStruct((_EMB_D, _SC_NPAD), jnp.float32),
        mesh=plsc.VectorSubcoreMesh(core_axis_name="c", subcore_axis_name="s"),
        compiler_params=pltpu.CompilerParams(needs_layout_passes=False),
        scratch_types=[
            pltpu.VMEM((_SC_BPW,), jnp.int32),
            pltpu.VMEM((_MAX_Z * _EMB_D,), jnp.float32),
            pltpu.VMEM((_EMB_D, _SC_BPW), jnp.float32),
        ],
    )
    def _emb_sc(idx_hbm, table_hbm, out_hbm, idx_v, table_v, rows_v):
        wid = lax.axis_index("s") * 2 + lax.axis_index("c")
        base = wid * _SC_BPW
        pltpu.sync_copy(idx_hbm.at[pl.ds(base, _SC_BPW)], idx_v)
        pltpu.sync_copy(table_hbm, table_v)

        @plsc.parallel_loop(0, _SC_GRP, step=1, unroll=4)
        def _gather_loop(g):
            src = idx_v[pl.ds(g * 16, 16)] * _EMB_D
            for j in range(_EMB_D):
                vals = plsc.load_gather(table_v, [src + j])
                rows_v[j, pl.ds(g * 16, 16)] = vals

        @pl.when(wid < _SC_NW - 1)
        def _():
            pltpu.sync_copy(rows_v, out_hbm.at[:, pl.ds(base, _SC_BPW)])

        @pl.when(wid == _SC_NW - 1)
        def _():
            pltpu.sync_copy(rows_v.at[:, pl.ds(0, _SC_LAST)],
                            out_hbm.at[:, pl.ds(base, _SC_LAST)])

    return _emb_sc


def _emb_gather(atomic_numbers, embedding_table):
    pad = _SC_NW * _SC_BPW - atomic_numbers.shape[0]
    idx = jnp.concatenate([atomic_numbers, jnp.zeros((pad,), jnp.int32)])
    emb_t = _build_emb_sc()(idx, embedding_table.reshape(-1))
    return emb_t[:, :_N_ATOMS].T


def kernel(d_ij, atomic_numbers, embedding_table):
    f_ij, f_cutoff = _rbf_tc(d_ij.reshape(-1))
    atomic_embedding = _emb_gather(atomic_numbers, embedding_table)
    return (f_ij, f_cutoff, atomic_embedding)
